# Initial kernel scaffold; baseline (speedup 1.0000x reference)
import functools

import jax
import jax.numpy as jnp
from jax import lax
from jax.experimental import pallas as pl
from jax.experimental.pallas import tpu as pltpu
from jax.experimental.pallas import tpu_sc as plsc

N = 10000
E = 320000
D = 128
ED = 16

NC = 2
NS = 16
NW = NC * NS
EW = E // NW
CH = 80
ROWS_A = 624
TAIL_ROWS = N - NS * ROWS_A

_mesh = plsc.VectorSubcoreMesh(core_axis_name="c", subcore_axis_name="s")


@functools.partial(
    pl.kernel,
    mesh=_mesh,
    out_type=[
        jax.ShapeDtypeStruct((E, D), jnp.float32),
        jax.ShapeDtypeStruct((E, D), jnp.float32),
    ],
    scratch_types=[
        pltpu.VMEM((CH,), jnp.int32),
        pltpu.VMEM((CH, D), jnp.float32),
        pltpu.VMEM((CH,), jnp.int32),
        pltpu.VMEM((CH, D), jnp.float32),
    ],
)
def _sc_gather(h_hbm, src_hbm, dst_hbm, hs_hbm, hd_hbm, si_v, srow_v, di_v, drow_v):
    wid = lax.axis_index("s") * NC + lax.axis_index("c")
    base = wid * EW

    @pl.loop(0, EW, step=CH)
    def _(off):
        b = base + off
        pltpu.sync_copy(src_hbm.at[pl.ds(b, CH)], si_v)
        pltpu.sync_copy(h_hbm.at[si_v], srow_v)
        pltpu.sync_copy(srow_v, hs_hbm.at[pl.ds(b, CH)])
        pltpu.sync_copy(dst_hbm.at[pl.ds(b, CH)], di_v)
        pltpu.sync_copy(h_hbm.at[di_v], drow_v)
        pltpu.sync_copy(drow_v, hd_hbm.at[pl.ds(b, CH)])


def _mlp_body(hs_ref, hd_ref, e_ref, ws_ref, wd_ref, we_ref, bc_ref, w2_ref, b2_ref, out_ref):
    u = (
        jnp.dot(hs_ref[...], ws_ref[...], preferred_element_type=jnp.float32)
        + jnp.dot(hd_ref[...], wd_ref[...], preferred_element_type=jnp.float32)
        + jnp.dot(e_ref[...], we_ref[...], preferred_element_type=jnp.float32)
        + bc_ref[...]
    )
    pre = u[:, :D]
    gate_in = u[:, D:]
    hidden = pre * jax.nn.sigmoid(pre)
    msg = jnp.dot(hidden, w2_ref[...], preferred_element_type=jnp.float32) + b2_ref[...]
    out_ref[...] = msg * jax.nn.sigmoid(gate_in)


def _edge_mlp(hs, hd, e, ws, wd, we, bc, w2, b2, block=4000):
    grid = (E // block,)
    full = lambda shape: pl.BlockSpec(shape, lambda i: (0, 0))
    return pl.pallas_call(
        _mlp_body,
        grid=grid,
        in_specs=[
            pl.BlockSpec((block, D), lambda i: (i, 0)),
            pl.BlockSpec((block, D), lambda i: (i, 0)),
            pl.BlockSpec((block, ED), lambda i: (i, 0)),
            full((D, 2 * D)),
            full((D, 2 * D)),
            full((ED, 2 * D)),
            full((1, 2 * D)),
            full((D, D)),
            full((1, D)),
        ],
        out_specs=pl.BlockSpec((block, D), lambda i: (i, 0)),
        out_shape=jax.ShapeDtypeStruct((E, D), jnp.float32),
    )(hs, hd, e, ws, wd, we, bc, w2, b2)


@functools.partial(
    pl.kernel,
    mesh=_mesh,
    out_type=[
        jax.ShapeDtypeStruct((NC, N, D), jnp.float32),
        jax.ShapeDtypeStruct((NC, N, ED), jnp.float32),
    ],
    scratch_types=[
        pltpu.VMEM((CH,), jnp.int32),
        pltpu.VMEM((CH, D), jnp.float32),
        pltpu.VMEM((CH, ED), jnp.float32),
        pltpu.VMEM_SHARED((N, D), jnp.float32),
        pltpu.VMEM_SHARED((N, ED), jnp.float32),
    ],
)
def _sc_scatter(msg_hbm, dst_hbm, z_d_hbm, z_e_hbm, ones_hbm, agg_hbm, deg_hbm,
                di_v, msg_v, ones_v, acc_sh, deg_sh):
    cid = lax.axis_index("c")
    sid = lax.axis_index("s")
    wid = sid * NC + cid
    base = wid * EW

    r0 = sid * ROWS_A
    pltpu.sync_copy(z_d_hbm.at[pl.ds(r0, ROWS_A)], acc_sh.at[pl.ds(r0, ROWS_A)])
    pltpu.sync_copy(z_e_hbm.at[pl.ds(r0, ROWS_A)], deg_sh.at[pl.ds(r0, ROWS_A)])

    @pl.when(sid == 0)
    def _():
        t0 = NS * ROWS_A
        pltpu.sync_copy(z_d_hbm.at[pl.ds(t0, TAIL_ROWS)], acc_sh.at[pl.ds(t0, TAIL_ROWS)])
        pltpu.sync_copy(z_e_hbm.at[pl.ds(t0, TAIL_ROWS)], deg_sh.at[pl.ds(t0, TAIL_ROWS)])

    pltpu.sync_copy(ones_hbm, ones_v)
    plsc.subcore_barrier()

    @pl.loop(0, EW, step=CH)
    def _(off):
        b = base + off
        pltpu.sync_copy(dst_hbm.at[pl.ds(b, CH)], di_v)
        pltpu.sync_copy(msg_hbm.at[pl.ds(b, CH)], msg_v)
        pltpu.sync_copy(msg_v, acc_sh.at[di_v], add=True)
        pltpu.sync_copy(ones_v, deg_sh.at[di_v], add=True)

    plsc.subcore_barrier()

    pltpu.sync_copy(acc_sh.at[pl.ds(r0, ROWS_A)], agg_hbm.at[cid, pl.ds(r0, ROWS_A)])
    pltpu.sync_copy(deg_sh.at[pl.ds(r0, ROWS_A)], deg_hbm.at[cid, pl.ds(r0, ROWS_A)])

    @pl.when(sid == 0)
    def _():
        t0 = NS * ROWS_A
        pltpu.sync_copy(acc_sh.at[pl.ds(t0, TAIL_ROWS)], agg_hbm.at[cid, pl.ds(t0, TAIL_ROWS)])
        pltpu.sync_copy(deg_sh.at[pl.ds(t0, TAIL_ROWS)], deg_hbm.at[cid, pl.ds(t0, TAIL_ROWS)])


def _node_body(h_ref, agg_ref, deg_ref, wa_ref, wb_ref, bu_ref, out_ref):
    agg = agg_ref[0] + agg_ref[1]
    deg = deg_ref[0][:, :1] + deg_ref[1][:, :1]
    aggn = agg / jnp.maximum(deg, 1.0)
    z = (
        jnp.dot(h_ref[...], wa_ref[...], preferred_element_type=jnp.float32)
        + jnp.dot(aggn, wb_ref[...], preferred_element_type=jnp.float32)
        + bu_ref[...]
    )
    out_ref[...] = z * jax.nn.sigmoid(z)


def _node_update(h, aggp, degp, wa, wb, bu, block=2000):
    grid = (N // block,)
    full = lambda shape: pl.BlockSpec(shape, lambda i: tuple(0 for _ in shape))
    return pl.pallas_call(
        _node_body,
        grid=grid,
        in_specs=[
            pl.BlockSpec((block, D), lambda i: (i, 0)),
            pl.BlockSpec((NC, block, D), lambda i: (0, i, 0)),
            pl.BlockSpec((NC, block, ED), lambda i: (0, i, 0)),
            full((D, D)),
            full((D, D)),
            full((1, D)),
        ],
        out_specs=pl.BlockSpec((block, D), lambda i: (i, 0)),
        out_shape=jax.ShapeDtypeStruct((N, D), jnp.float32),
    )(h, aggp, degp, wa, wb, bu)


def kernel(h, edge_index, e, W1, b1, W2, b2, Wg, bg, Wu, bu):
    src = edge_index[0]
    dst = edge_index[1]

    ws = jnp.concatenate([W1[:D], Wg[:D]], axis=1)
    wd = jnp.concatenate([W1[D:2 * D], Wg[D:2 * D]], axis=1)
    we = jnp.concatenate([W1[2 * D:], Wg[2 * D:]], axis=1)
    bc = jnp.concatenate([b1, bg])[None, :]

    hs, hd = _sc_gather(h, src, dst)
    msg = _edge_mlp(hs, hd, e, ws, wd, we, bc, W2, b2[None, :])

    z_d = jnp.zeros((N, D), jnp.float32)
    z_e = jnp.zeros((N, ED), jnp.float32)
    ones = jnp.ones((CH, ED), jnp.float32)
    aggp, degp = _sc_scatter(msg, dst, z_d, z_e, ones)

    return _node_update(h, aggp, degp, Wu[:D], Wu[D:], bu[None, :])


# speedup vs baseline: 2.7953x; 2.7953x over previous
import functools

import jax
import jax.numpy as jnp
from jax import lax
from jax.experimental import pallas as pl
from jax.experimental.pallas import tpu as pltpu
from jax.experimental.pallas import tpu_sc as plsc

N = 10000
E = 320000
D = 128
ED = 16

NC = 2
NS = 16
NW = NC * NS
EW = E // NW
CH = 80
ROWS_A = 624
TAIL_ROWS = N - NS * ROWS_A


@functools.cache
def _make_sc_gather():
    mesh = plsc.VectorSubcoreMesh(core_axis_name="c", subcore_axis_name="s")

    @functools.partial(
        pl.kernel,
        mesh=mesh,
        out_type=[
            jax.ShapeDtypeStruct((E, D), jnp.float32),
            jax.ShapeDtypeStruct((E, D), jnp.float32),
            jax.ShapeDtypeStruct((NC, N, D), jnp.float32),
        ],
        scratch_types=[
            pltpu.VMEM((CH,), jnp.int32),
            pltpu.VMEM((CH, D), jnp.float32),
            pltpu.VMEM((CH,), jnp.int32),
            pltpu.VMEM((CH, D), jnp.float32),
            pltpu.VMEM((CH, D), jnp.float32),
            pltpu.VMEM_SHARED((N, D), jnp.float32),
        ],
    )
    def _sc_gather(h_hbm, src_hbm, dst_hbm, z_e_hbm, ones_hbm,
                   hs_hbm, hd_hbm, deg_hbm,
                   si_v, srow_v, di_v, drow_v, ones_v, deg_sh):
        cid = lax.axis_index("c")
        sid = lax.axis_index("s")
        wid = sid * NC + cid
        base = wid * EW

        r0 = sid * ROWS_A
        pltpu.sync_copy(z_e_hbm.at[pl.ds(r0, ROWS_A)], deg_sh.at[pl.ds(r0, ROWS_A)])

        @pl.when(sid == 0)
        def _():
            t0 = NS * ROWS_A
            pltpu.sync_copy(z_e_hbm.at[pl.ds(t0, TAIL_ROWS)], deg_sh.at[pl.ds(t0, TAIL_ROWS)])

        pltpu.sync_copy(ones_hbm, ones_v)
        plsc.subcore_barrier()

        @pl.loop(0, EW, step=CH)
        def _(off):
            b = base + off
            pltpu.sync_copy(src_hbm.at[pl.ds(b, CH)], si_v)
            pltpu.sync_copy(h_hbm.at[si_v], srow_v)
            pltpu.sync_copy(srow_v, hs_hbm.at[pl.ds(b, CH)])
            pltpu.sync_copy(dst_hbm.at[pl.ds(b, CH)], di_v)
            pltpu.sync_copy(h_hbm.at[di_v], drow_v)
            pltpu.sync_copy(drow_v, hd_hbm.at[pl.ds(b, CH)])
            pltpu.sync_copy(ones_v, deg_sh.at[di_v], add=True)

        plsc.subcore_barrier()

        pltpu.sync_copy(deg_sh.at[pl.ds(r0, ROWS_A)], deg_hbm.at[cid].at[pl.ds(r0, ROWS_A)])

        @pl.when(sid == 0)
        def _():
            t0 = NS * ROWS_A
            pltpu.sync_copy(deg_sh.at[pl.ds(t0, TAIL_ROWS)], deg_hbm.at[cid].at[pl.ds(t0, TAIL_ROWS)])

    return _sc_gather


def _mlp_body(hs_ref, hd_ref, e_ref, ws_ref, wd_ref, we_ref, bc_ref, w2_ref, b2_ref, out_ref):
    u = (
        jnp.dot(hs_ref[...], ws_ref[...], preferred_element_type=jnp.float32)
        + jnp.dot(hd_ref[...], wd_ref[...], preferred_element_type=jnp.float32)
        + jnp.dot(e_ref[...], we_ref[...], preferred_element_type=jnp.float32)
        + bc_ref[...]
    )
    pre = u[:, :D]
    gate_in = u[:, D:]
    hidden = pre * jax.nn.sigmoid(pre)
    msg = jnp.dot(hidden, w2_ref[...], preferred_element_type=jnp.float32) + b2_ref[...]
    out_ref[...] = msg * jax.nn.sigmoid(gate_in)


def _edge_mlp(hs, hd, e, ws, wd, we, bc, w2, b2, block=4000):
    grid = (E // block,)
    full = lambda shape: pl.BlockSpec(shape, lambda i: (0, 0))
    return pl.pallas_call(
        _mlp_body,
        grid=grid,
        in_specs=[
            pl.BlockSpec((block, D), lambda i: (i, 0)),
            pl.BlockSpec((block, D), lambda i: (i, 0)),
            pl.BlockSpec((block, ED), lambda i: (i, 0)),
            full((D, 2 * D)),
            full((D, 2 * D)),
            full((ED, 2 * D)),
            full((1, 2 * D)),
            full((D, D)),
            full((1, D)),
        ],
        out_specs=pl.BlockSpec((block, D), lambda i: (i, 0)),
        out_shape=jax.ShapeDtypeStruct((E, D), jnp.float32),
    )(hs, hd, e, ws, wd, we, bc, w2, b2)


@functools.cache
def _make_sc_scatter():
    mesh = plsc.VectorSubcoreMesh(core_axis_name="c", subcore_axis_name="s")

    @functools.partial(
        pl.kernel,
        mesh=mesh,
        out_type=jax.ShapeDtypeStruct((NC, N, D), jnp.float32),
        scratch_types=[
            pltpu.VMEM((CH,), jnp.int32),
            pltpu.VMEM((CH, D), jnp.float32),
            pltpu.VMEM_SHARED((N, D), jnp.float32),
        ],
    )
    def _sc_scatter(msg_hbm, dst_hbm, z_d_hbm, agg_hbm, di_v, msg_v, acc_sh):
        cid = lax.axis_index("c")
        sid = lax.axis_index("s")
        wid = sid * NC + cid
        base = wid * EW

        r0 = sid * ROWS_A
        pltpu.sync_copy(z_d_hbm.at[pl.ds(r0, ROWS_A)], acc_sh.at[pl.ds(r0, ROWS_A)])

        @pl.when(sid == 0)
        def _():
            t0 = NS * ROWS_A
            pltpu.sync_copy(z_d_hbm.at[pl.ds(t0, TAIL_ROWS)], acc_sh.at[pl.ds(t0, TAIL_ROWS)])

        plsc.subcore_barrier()

        @pl.loop(0, EW, step=CH)
        def _(off):
            b = base + off
            pltpu.sync_copy(dst_hbm.at[pl.ds(b, CH)], di_v)
            pltpu.sync_copy(msg_hbm.at[pl.ds(b, CH)], msg_v)
            pltpu.sync_copy(msg_v, acc_sh.at[di_v], add=True)

        plsc.subcore_barrier()

        pltpu.sync_copy(acc_sh.at[pl.ds(r0, ROWS_A)], agg_hbm.at[cid].at[pl.ds(r0, ROWS_A)])

        @pl.when(sid == 0)
        def _():
            t0 = NS * ROWS_A
            pltpu.sync_copy(acc_sh.at[pl.ds(t0, TAIL_ROWS)], agg_hbm.at[cid].at[pl.ds(t0, TAIL_ROWS)])

    return _sc_scatter


def _node_body(h_ref, agg_ref, deg_ref, wa_ref, wb_ref, bu_ref, out_ref):
    agg = agg_ref[0] + agg_ref[1]
    deg = deg_ref[0][:, :1] + deg_ref[1][:, :1]
    aggn = agg / jnp.maximum(deg, 1.0)
    z = (
        jnp.dot(h_ref[...], wa_ref[...], preferred_element_type=jnp.float32)
        + jnp.dot(aggn, wb_ref[...], preferred_element_type=jnp.float32)
        + bu_ref[...]
    )
    out_ref[...] = z * jax.nn.sigmoid(z)


def _node_update(h, aggp, degp, wa, wb, bu, block=2000):
    grid = (N // block,)
    full = lambda shape: pl.BlockSpec(shape, lambda i: tuple(0 for _ in shape))
    return pl.pallas_call(
        _node_body,
        grid=grid,
        in_specs=[
            pl.BlockSpec((block, D), lambda i: (i, 0)),
            pl.BlockSpec((NC, block, D), lambda i: (0, i, 0)),
            pl.BlockSpec((NC, block, D), lambda i: (0, i, 0)),
            full((D, D)),
            full((D, D)),
            full((1, D)),
        ],
        out_specs=pl.BlockSpec((block, D), lambda i: (i, 0)),
        out_shape=jax.ShapeDtypeStruct((N, D), jnp.float32),
    )(h, aggp, degp, wa, wb, bu)


def kernel(h, edge_index, e, W1, b1, W2, b2, Wg, bg, Wu, bu):
    src = edge_index[0]
    dst = edge_index[1]

    ws = jnp.concatenate([W1[:D], Wg[:D]], axis=1)
    wd = jnp.concatenate([W1[D:2 * D], Wg[D:2 * D]], axis=1)
    we = jnp.concatenate([W1[2 * D:], Wg[2 * D:]], axis=1)
    bc = jnp.concatenate([b1, bg])[None, :]

    z_d = jnp.zeros((N, D), jnp.float32)
    ones = jnp.ones((CH, D), jnp.float32)
    hs, hd, degp = _make_sc_gather()(h, src, dst, z_d, ones)

    msg = _edge_mlp(hs, hd, e, ws, wd, we, bc, W2, b2[None, :])

    aggp = _make_sc_scatter()(msg, dst, z_d)

    return _node_update(h, aggp, degp, Wu[:D], Wu[D:], bu[None, :])


# speedup vs baseline: 4.2994x; 1.5381x over previous
import functools

import jax
import jax.numpy as jnp
from jax import lax
from jax.experimental import pallas as pl
from jax.experimental.pallas import tpu as pltpu
from jax.experimental.pallas import tpu_sc as plsc

N = 10000
E = 320000
D = 128
ED = 16

NC = 2
NS = 16
NW = NC * NS
EW = E // NW
CH = 80
K = 5
G = K * CH
CHS = 40
GS = K * CHS
ROWS_A = 624
TAIL_ROWS = N - NS * ROWS_A


@functools.cache
def _make_sc_gather():
    mesh = plsc.VectorSubcoreMesh(core_axis_name="c", subcore_axis_name="s")

    @functools.partial(
        pl.kernel,
        mesh=mesh,
        out_type=[
            jax.ShapeDtypeStruct((E, D), jnp.float32),
            jax.ShapeDtypeStruct((E, D), jnp.float32),
        ],
        scratch_types=[
            pltpu.VMEM((K, CH), jnp.int32),
            pltpu.VMEM((G, D), jnp.float32),
            pltpu.VMEM((K, CH), jnp.int32),
            pltpu.VMEM((G, D), jnp.float32),
            pltpu.SemaphoreType.DMA,
            pltpu.SemaphoreType.DMA,
            pltpu.SemaphoreType.DMA,
        ],
    )
    def _sc_gather(h_hbm, src_hbm, dst_hbm, hs_hbm, hd_hbm,
                   si_v, srow_v, di_v, drow_v, semi, semg, semw):
        cid = lax.axis_index("c")
        sid = lax.axis_index("s")
        wid = sid * NC + cid
        base = wid * EW

        @pl.loop(0, EW, step=G)
        def _(off):
            b = base + off
            hi = [pltpu.async_copy(src_hbm.at[pl.ds(b + j * CH, CH)], si_v.at[j], semi)
                  for j in range(K)]
            hi += [pltpu.async_copy(dst_hbm.at[pl.ds(b + j * CH, CH)], di_v.at[j], semi)
                   for j in range(K)]
            for hh in hi:
                hh.wait()
            hg = [pltpu.async_copy(h_hbm.at[si_v.at[j]], srow_v.at[pl.ds(j * CH, CH)], semg)
                  for j in range(K)]
            hg += [pltpu.async_copy(h_hbm.at[di_v.at[j]], drow_v.at[pl.ds(j * CH, CH)], semg)
                   for j in range(K)]
            for hh in hg:
                hh.wait()
            hw = [pltpu.async_copy(srow_v.at[pl.ds(j * CH, CH)], hs_hbm.at[pl.ds(b + j * CH, CH)], semw)
                  for j in range(K)]
            hw += [pltpu.async_copy(drow_v.at[pl.ds(j * CH, CH)], hd_hbm.at[pl.ds(b + j * CH, CH)], semw)
                   for j in range(K)]
            for hh in hw:
                hh.wait()

    return _sc_gather


@functools.cache
def _make_sc_degree():
    mesh = plsc.VectorSubcoreMesh(core_axis_name="c", subcore_axis_name="s")

    @functools.partial(
        pl.kernel,
        mesh=mesh,
        out_type=jax.ShapeDtypeStruct((NC, N, D), jnp.float32),
        scratch_types=[
            pltpu.VMEM((K, CH), jnp.int32),
            pltpu.VMEM((CH, D), jnp.float32),
            pltpu.VMEM_SHARED((N, D), jnp.float32),
            pltpu.SemaphoreType.DMA,
            pltpu.SemaphoreType.DMA,
        ],
    )
    def _sc_degree(dst_hbm, z_d_hbm, ones_hbm, deg_hbm,
                   di_v, ones_v, deg_sh, semi, semw):
        cid = lax.axis_index("c")
        sid = lax.axis_index("s")
        wid = sid * NC + cid
        base = wid * EW

        r0 = sid * ROWS_A
        pltpu.sync_copy(z_d_hbm.at[pl.ds(r0, ROWS_A)], deg_sh.at[pl.ds(r0, ROWS_A)])

        @pl.when(sid == 0)
        def _():
            t0 = NS * ROWS_A
            pltpu.sync_copy(z_d_hbm.at[pl.ds(t0, TAIL_ROWS)], deg_sh.at[pl.ds(t0, TAIL_ROWS)])

        pltpu.sync_copy(ones_hbm, ones_v)
        plsc.subcore_barrier()

        @pl.loop(0, EW, step=G)
        def _(off):
            b = base + off
            hi = [pltpu.async_copy(dst_hbm.at[pl.ds(b + j * CH, CH)], di_v.at[j], semi)
                  for j in range(K)]
            for hh in hi:
                hh.wait()
            hw = [pltpu.async_copy(ones_v, deg_sh.at[di_v.at[j]], semw, add=True)
                  for j in range(K)]
            for hh in hw:
                hh.wait()

        plsc.subcore_barrier()

        pltpu.sync_copy(deg_sh.at[pl.ds(r0, ROWS_A)], deg_hbm.at[cid].at[pl.ds(r0, ROWS_A)])

        @pl.when(sid == 0)
        def _():
            t0 = NS * ROWS_A
            pltpu.sync_copy(deg_sh.at[pl.ds(t0, TAIL_ROWS)], deg_hbm.at[cid].at[pl.ds(t0, TAIL_ROWS)])

    return _sc_degree


def _mlp_body(hs_ref, hd_ref, e_ref, ws_ref, wd_ref, we_ref, bc_ref, w2_ref, b2_ref, out_ref):
    u = (
        jnp.dot(hs_ref[...], ws_ref[...], preferred_element_type=jnp.float32)
        + jnp.dot(hd_ref[...], wd_ref[...], preferred_element_type=jnp.float32)
        + jnp.dot(e_ref[...], we_ref[...], preferred_element_type=jnp.float32)
        + bc_ref[...]
    )
    pre = u[:, :D]
    gate_in = u[:, D:]
    hidden = pre * jax.nn.sigmoid(pre)
    msg = jnp.dot(hidden, w2_ref[...], preferred_element_type=jnp.float32) + b2_ref[...]
    out_ref[...] = msg * jax.nn.sigmoid(gate_in)


def _edge_mlp(hs, hd, e, ws, wd, we, bc, w2, b2, block=4000):
    grid = (E // block,)
    full = lambda shape: pl.BlockSpec(shape, lambda i: (0, 0))
    return pl.pallas_call(
        _mlp_body,
        grid=grid,
        in_specs=[
            pl.BlockSpec((block, D), lambda i: (i, 0)),
            pl.BlockSpec((block, D), lambda i: (i, 0)),
            pl.BlockSpec((block, ED), lambda i: (i, 0)),
            full((D, 2 * D)),
            full((D, 2 * D)),
            full((ED, 2 * D)),
            full((1, 2 * D)),
            full((D, D)),
            full((1, D)),
        ],
        out_specs=pl.BlockSpec((block, D), lambda i: (i, 0)),
        out_shape=jax.ShapeDtypeStruct((E, D), jnp.float32),
    )(hs, hd, e, ws, wd, we, bc, w2, b2)


@functools.cache
def _make_sc_scatter():
    mesh = plsc.VectorSubcoreMesh(core_axis_name="c", subcore_axis_name="s")

    @functools.partial(
        pl.kernel,
        mesh=mesh,
        out_type=jax.ShapeDtypeStruct((NC, N, D), jnp.float32),
        scratch_types=[
            pltpu.VMEM((K, CHS), jnp.int32),
            pltpu.VMEM((GS, D), jnp.float32),
            pltpu.VMEM_SHARED((N, D), jnp.float32),
            pltpu.SemaphoreType.DMA,
            pltpu.SemaphoreType.DMA,
        ],
    )
    def _sc_scatter(msg_hbm, dst_hbm, z_d_hbm, agg_hbm, di_v, msg_v, acc_sh, semi, sema):
        cid = lax.axis_index("c")
        sid = lax.axis_index("s")
        wid = sid * NC + cid
        base = wid * EW

        r0 = sid * ROWS_A
        pltpu.sync_copy(z_d_hbm.at[pl.ds(r0, ROWS_A)], acc_sh.at[pl.ds(r0, ROWS_A)])

        @pl.when(sid == 0)
        def _():
            t0 = NS * ROWS_A
            pltpu.sync_copy(z_d_hbm.at[pl.ds(t0, TAIL_ROWS)], acc_sh.at[pl.ds(t0, TAIL_ROWS)])

        plsc.subcore_barrier()

        @pl.loop(0, EW, step=GS)
        def _(off):
            b = base + off
            hi = [pltpu.async_copy(dst_hbm.at[pl.ds(b + j * CHS, CHS)], di_v.at[j], semi)
                  for j in range(K)]
            hi += [pltpu.async_copy(msg_hbm.at[pl.ds(b + j * CHS, CHS)],
                                    msg_v.at[pl.ds(j * CHS, CHS)], semi)
                   for j in range(K)]
            for hh in hi:
                hh.wait()
            ha = [pltpu.async_copy(msg_v.at[pl.ds(j * CHS, CHS)], acc_sh.at[di_v.at[j]], sema, add=True)
                  for j in range(K)]
            for hh in ha:
                hh.wait()

        plsc.subcore_barrier()

        pltpu.sync_copy(acc_sh.at[pl.ds(r0, ROWS_A)], agg_hbm.at[cid].at[pl.ds(r0, ROWS_A)])

        @pl.when(sid == 0)
        def _():
            t0 = NS * ROWS_A
            pltpu.sync_copy(acc_sh.at[pl.ds(t0, TAIL_ROWS)], agg_hbm.at[cid].at[pl.ds(t0, TAIL_ROWS)])

    return _sc_scatter


def _node_body(h_ref, agg_ref, deg_ref, wa_ref, wb_ref, bu_ref, out_ref):
    agg = agg_ref[0] + agg_ref[1]
    deg = deg_ref[0][:, :1] + deg_ref[1][:, :1]
    aggn = agg / jnp.maximum(deg, 1.0)
    z = (
        jnp.dot(h_ref[...], wa_ref[...], preferred_element_type=jnp.float32)
        + jnp.dot(aggn, wb_ref[...], preferred_element_type=jnp.float32)
        + bu_ref[...]
    )
    out_ref[...] = z * jax.nn.sigmoid(z)


def _node_update(h, aggp, degp, wa, wb, bu, block=2000):
    grid = (N // block,)
    full = lambda shape: pl.BlockSpec(shape, lambda i: tuple(0 for _ in shape))
    return pl.pallas_call(
        _node_body,
        grid=grid,
        in_specs=[
            pl.BlockSpec((block, D), lambda i: (i, 0)),
            pl.BlockSpec((NC, block, D), lambda i: (0, i, 0)),
            pl.BlockSpec((NC, block, D), lambda i: (0, i, 0)),
            full((D, D)),
            full((D, D)),
            full((1, D)),
        ],
        out_specs=pl.BlockSpec((block, D), lambda i: (i, 0)),
        out_shape=jax.ShapeDtypeStruct((N, D), jnp.float32),
    )(h, aggp, degp, wa, wb, bu)


def kernel(h, edge_index, e, W1, b1, W2, b2, Wg, bg, Wu, bu):
    src = edge_index[0]
    dst = edge_index[1]

    ws = jnp.concatenate([W1[:D], Wg[:D]], axis=1)
    wd = jnp.concatenate([W1[D:2 * D], Wg[D:2 * D]], axis=1)
    we = jnp.concatenate([W1[2 * D:], Wg[2 * D:]], axis=1)
    bc = jnp.concatenate([b1, bg])[None, :]

    z_d = jnp.zeros((N, D), jnp.float32)
    ones = jnp.ones((CH, D), jnp.float32)
    hs, hd = _make_sc_gather()(h, src, dst)
    degp = _make_sc_degree()(dst, z_d, ones)

    msg = _edge_mlp(hs, hd, e, ws, wd, we, bc, W2, b2[None, :])

    aggp = _make_sc_scatter()(msg, dst, z_d)

    return _node_update(h, aggp, degp, Wu[:D], Wu[D:], bu[None, :])


# speedup vs baseline: 4.4209x; 1.0283x over previous
import functools

import jax
import jax.numpy as jnp
from jax import lax
from jax.experimental import pallas as pl
from jax.experimental.pallas import tpu as pltpu
from jax.experimental.pallas import tpu_sc as plsc

N = 10000
E = 320000
D = 128
ED = 16

NC = 2
NS = 16
NW = NC * NS
EW = E // NW
CH = 80
K = 5
G = K * CH
CHS = 40
GS = K * CHS
ROWS_A = 624
TAIL_ROWS = N - NS * ROWS_A


@functools.cache
def _make_sc_gather(ne):
    mesh = plsc.VectorSubcoreMesh(core_axis_name="c", subcore_axis_name="s")
    ew = ne // NW

    @functools.partial(
        pl.kernel,
        mesh=mesh,
        out_type=[
            jax.ShapeDtypeStruct((ne, D), jnp.float32),
            jax.ShapeDtypeStruct((ne, D), jnp.float32),
        ],
        scratch_types=[
            pltpu.VMEM((K, CHS), jnp.int32),
            pltpu.VMEM((GS, D), jnp.float32),
            pltpu.VMEM((K, CHS), jnp.int32),
            pltpu.VMEM((GS, D), jnp.float32),
            pltpu.SemaphoreType.DMA,
            pltpu.SemaphoreType.DMA,
            pltpu.SemaphoreType.DMA,
        ],
    )
    def _sc_gather(h_hbm, src_hbm, dst_hbm, hs_hbm, hd_hbm,
                   si_v, srow_v, di_v, drow_v, semi, semg, semw):
        cid = lax.axis_index("c")
        sid = lax.axis_index("s")
        wid = sid * NC + cid
        base = wid * ew

        @pl.loop(0, ew, step=GS)
        def _(off):
            b = base + off
            hi = [pltpu.async_copy(src_hbm.at[pl.ds(b + j * CHS, CHS)], si_v.at[j], semi)
                  for j in range(K)]
            hi += [pltpu.async_copy(dst_hbm.at[pl.ds(b + j * CHS, CHS)], di_v.at[j], semi)
                   for j in range(K)]
            for hh in hi:
                hh.wait()
            hg = [pltpu.async_copy(h_hbm.at[si_v.at[j]], srow_v.at[pl.ds(j * CHS, CHS)], semg)
                  for j in range(K)]
            hg += [pltpu.async_copy(h_hbm.at[di_v.at[j]], drow_v.at[pl.ds(j * CHS, CHS)], semg)
                   for j in range(K)]
            for hh in hg:
                hh.wait()
            hw = [pltpu.async_copy(srow_v.at[pl.ds(j * CHS, CHS)], hs_hbm.at[pl.ds(b + j * CHS, CHS)], semw)
                  for j in range(K)]
            hw += [pltpu.async_copy(drow_v.at[pl.ds(j * CHS, CHS)], hd_hbm.at[pl.ds(b + j * CHS, CHS)], semw)
                   for j in range(K)]
            for hh in hw:
                hh.wait()

    return _sc_gather


@functools.cache
def _make_sc_degree():
    mesh = plsc.VectorSubcoreMesh(core_axis_name="c", subcore_axis_name="s")

    @functools.partial(
        pl.kernel,
        mesh=mesh,
        out_type=jax.ShapeDtypeStruct((NC, N, D), jnp.float32),
        scratch_types=[
            pltpu.VMEM((K, CH), jnp.int32),
            pltpu.VMEM((CH, D), jnp.float32),
            pltpu.VMEM_SHARED((N, D), jnp.float32),
            pltpu.SemaphoreType.DMA,
            pltpu.SemaphoreType.DMA,
        ],
    )
    def _sc_degree(dst_hbm, z_d_hbm, ones_hbm, deg_hbm,
                   di_v, ones_v, deg_sh, semi, semw):
        cid = lax.axis_index("c")
        sid = lax.axis_index("s")
        wid = sid * NC + cid
        base = wid * EW

        r0 = sid * ROWS_A
        pltpu.sync_copy(z_d_hbm.at[pl.ds(r0, ROWS_A)], deg_sh.at[pl.ds(r0, ROWS_A)])

        @pl.when(sid == 0)
        def _():
            t0 = NS * ROWS_A
            pltpu.sync_copy(z_d_hbm.at[pl.ds(t0, TAIL_ROWS)], deg_sh.at[pl.ds(t0, TAIL_ROWS)])

        pltpu.sync_copy(ones_hbm, ones_v)
        plsc.subcore_barrier()

        @pl.loop(0, EW, step=G)
        def _(off):
            b = base + off
            hi = [pltpu.async_copy(dst_hbm.at[pl.ds(b + j * CH, CH)], di_v.at[j], semi)
                  for j in range(K)]
            for hh in hi:
                hh.wait()
            hw = [pltpu.async_copy(ones_v, deg_sh.at[di_v.at[j]], semw, add=True)
                  for j in range(K)]
            for hh in hw:
                hh.wait()

        plsc.subcore_barrier()

        pltpu.sync_copy(deg_sh.at[pl.ds(r0, ROWS_A)], deg_hbm.at[cid].at[pl.ds(r0, ROWS_A)])

        @pl.when(sid == 0)
        def _():
            t0 = NS * ROWS_A
            pltpu.sync_copy(deg_sh.at[pl.ds(t0, TAIL_ROWS)], deg_hbm.at[cid].at[pl.ds(t0, TAIL_ROWS)])

    return _sc_degree


def _mlp_body(hs_ref, hd_ref, e_ref, ws_ref, wd_ref, we_ref, bc_ref, w2_ref, b2_ref, out_ref):
    u = (
        jnp.dot(hs_ref[...], ws_ref[...], preferred_element_type=jnp.float32)
        + jnp.dot(hd_ref[...], wd_ref[...], preferred_element_type=jnp.float32)
        + jnp.dot(e_ref[...], we_ref[...], preferred_element_type=jnp.float32)
        + bc_ref[...]
    )
    pre = u[:, :D]
    gate_in = u[:, D:]
    hidden = pre * jax.nn.sigmoid(pre)
    msg = jnp.dot(hidden, w2_ref[...], preferred_element_type=jnp.float32) + b2_ref[...]
    out_ref[...] = msg * jax.nn.sigmoid(gate_in)


def _edge_mlp(hs, hd, e, ws, wd, we, bc, w2, b2, block=4000):
    grid = (hs.shape[0] // block,)
    full = lambda shape: pl.BlockSpec(shape, lambda i: (0, 0))
    return pl.pallas_call(
        _mlp_body,
        grid=grid,
        in_specs=[
            pl.BlockSpec((block, D), lambda i: (i, 0)),
            pl.BlockSpec((block, D), lambda i: (i, 0)),
            pl.BlockSpec((block, ED), lambda i: (i, 0)),
            full((D, 2 * D)),
            full((D, 2 * D)),
            full((ED, 2 * D)),
            full((1, 2 * D)),
            full((D, D)),
            full((1, D)),
        ],
        out_specs=pl.BlockSpec((block, D), lambda i: (i, 0)),
        out_shape=jax.ShapeDtypeStruct((hs.shape[0], D), jnp.float32),
    )(hs, hd, e, ws, wd, we, bc, w2, b2)


@functools.cache
def _make_sc_scatter(ne):
    mesh = plsc.VectorSubcoreMesh(core_axis_name="c", subcore_axis_name="s")
    ew = ne // NW

    @functools.partial(
        pl.kernel,
        mesh=mesh,
        out_type=jax.ShapeDtypeStruct((NC, N, D), jnp.float32),
        scratch_types=[
            pltpu.VMEM((K, CHS), jnp.int32),
            pltpu.VMEM((GS, D), jnp.float32),
            pltpu.VMEM_SHARED((N, D), jnp.float32),
            pltpu.SemaphoreType.DMA,
            pltpu.SemaphoreType.DMA,
        ],
    )
    def _sc_scatter(msg_hbm, dst_hbm, z_d_hbm, agg_hbm, di_v, msg_v, acc_sh, semi, sema):
        cid = lax.axis_index("c")
        sid = lax.axis_index("s")
        wid = sid * NC + cid
        base = wid * ew

        r0 = sid * ROWS_A
        pltpu.sync_copy(z_d_hbm.at[pl.ds(r0, ROWS_A)], acc_sh.at[pl.ds(r0, ROWS_A)])

        @pl.when(sid == 0)
        def _():
            t0 = NS * ROWS_A
            pltpu.sync_copy(z_d_hbm.at[pl.ds(t0, TAIL_ROWS)], acc_sh.at[pl.ds(t0, TAIL_ROWS)])

        plsc.subcore_barrier()

        @pl.loop(0, ew, step=GS)
        def _(off):
            b = base + off
            hi = [pltpu.async_copy(dst_hbm.at[pl.ds(b + j * CHS, CHS)], di_v.at[j], semi)
                  for j in range(K)]
            hi += [pltpu.async_copy(msg_hbm.at[pl.ds(b + j * CHS, CHS)],
                                    msg_v.at[pl.ds(j * CHS, CHS)], semi)
                   for j in range(K)]
            for hh in hi:
                hh.wait()
            ha = [pltpu.async_copy(msg_v.at[pl.ds(j * CHS, CHS)], acc_sh.at[di_v.at[j]], sema, add=True)
                  for j in range(K)]
            for hh in ha:
                hh.wait()

        plsc.subcore_barrier()

        pltpu.sync_copy(acc_sh.at[pl.ds(r0, ROWS_A)], agg_hbm.at[cid].at[pl.ds(r0, ROWS_A)])

        @pl.when(sid == 0)
        def _():
            t0 = NS * ROWS_A
            pltpu.sync_copy(acc_sh.at[pl.ds(t0, TAIL_ROWS)], agg_hbm.at[cid].at[pl.ds(t0, TAIL_ROWS)])

    return _sc_scatter


def _node_body(h_ref, agga_ref, aggb_ref, deg_ref, wa_ref, wb_ref, bu_ref, out_ref):
    agg = agga_ref[0] + agga_ref[1] + aggb_ref[0] + aggb_ref[1]
    deg = deg_ref[0][:, :1] + deg_ref[1][:, :1]
    aggn = agg / jnp.maximum(deg, 1.0)
    z = (
        jnp.dot(h_ref[...], wa_ref[...], preferred_element_type=jnp.float32)
        + jnp.dot(aggn, wb_ref[...], preferred_element_type=jnp.float32)
        + bu_ref[...]
    )
    out_ref[...] = z * jax.nn.sigmoid(z)


def _node_update(h, aggpa, aggpb, degp, wa, wb, bu, block=2000):
    grid = (N // block,)
    full = lambda shape: pl.BlockSpec(shape, lambda i: tuple(0 for _ in shape))
    return pl.pallas_call(
        _node_body,
        grid=grid,
        in_specs=[
            pl.BlockSpec((block, D), lambda i: (i, 0)),
            pl.BlockSpec((NC, block, D), lambda i: (0, i, 0)),
            pl.BlockSpec((NC, block, D), lambda i: (0, i, 0)),
            pl.BlockSpec((NC, block, D), lambda i: (0, i, 0)),
            full((D, D)),
            full((D, D)),
            full((1, D)),
        ],
        out_specs=pl.BlockSpec((block, D), lambda i: (i, 0)),
        out_shape=jax.ShapeDtypeStruct((N, D), jnp.float32),
    )(h, aggpa, aggpb, degp, wa, wb, bu)


def kernel(h, edge_index, e, W1, b1, W2, b2, Wg, bg, Wu, bu):
    src = edge_index[0]
    dst = edge_index[1]

    ws = jnp.concatenate([W1[:D], Wg[:D]], axis=1)
    wd = jnp.concatenate([W1[D:2 * D], Wg[D:2 * D]], axis=1)
    we = jnp.concatenate([W1[2 * D:], Wg[2 * D:]], axis=1)
    bc = jnp.concatenate([b1, bg])[None, :]

    z_d = jnp.zeros((N, D), jnp.float32)
    ones = jnp.ones((CH, D), jnp.float32)

    E2 = E // 2
    src_a, src_b = src[:E2], src[E2:]
    dst_a, dst_b = dst[:E2], dst[E2:]
    e_a, e_b = e[:E2], e[E2:]

    gather = _make_sc_gather(E2)
    scatter = _make_sc_scatter(E2)

    hs_a, hd_a = gather(h, src_a, dst_a)
    hs_b, hd_b = gather(h, src_b, dst_b)
    degp = _make_sc_degree()(dst, z_d, ones)

    msg_a = _edge_mlp(hs_a, hd_a, e_a, ws, wd, we, bc, W2, b2[None, :])
    msg_b = _edge_mlp(hs_b, hd_b, e_b, ws, wd, we, bc, W2, b2[None, :])

    aggpa = scatter(msg_a, dst_a, z_d)
    aggpb = scatter(msg_b, dst_b, z_d)

    return _node_update(h, aggpa, aggpb, degp, Wu[:D], Wu[D:], bu[None, :])


# speedup vs baseline: 4.5551x; 1.0304x over previous
import functools

import jax
import jax.numpy as jnp
from jax import lax
from jax.experimental import pallas as pl
from jax.experimental.pallas import tpu as pltpu
from jax.experimental.pallas import tpu_sc as plsc

N = 10000
E = 320000
D = 128
ED = 16

NC = 2
NS = 16
NW = NC * NS
EW = E // NW
CH = 80
K = 5
G = K * CH
CHS = 40
GS = K * CHS
CHB = 64
GB = K * CHB
ROWS_A = 624
TAIL_ROWS = N - NS * ROWS_A


DP = D // 2


@functools.cache
def _make_sc_gather(ne):
    mesh = plsc.VectorSubcoreMesh(core_axis_name="c", subcore_axis_name="s")
    ew = ne // NW
    full_groups = (ew // G) * G
    tail = ew - full_groups

    @functools.partial(
        pl.kernel,
        mesh=mesh,
        out_type=[
            jax.ShapeDtypeStruct((ne, D), jnp.float32),
            jax.ShapeDtypeStruct((ne, D), jnp.float32),
        ],
        scratch_types=[
            pltpu.VMEM((K, CH), jnp.int32),
            pltpu.VMEM((G, D), jnp.float32),
            pltpu.VMEM((K, CH), jnp.int32),
            pltpu.VMEM((G, D), jnp.float32),
            pltpu.SemaphoreType.DMA,
            pltpu.SemaphoreType.DMA,
            pltpu.SemaphoreType.DMA,
        ],
    )
    def _sc_gather(h_hbm, src_hbm, dst_hbm, hs_hbm, hd_hbm,
                   si_v, srow_v, di_v, drow_v, semi, semg, semw):
        cid = lax.axis_index("c")
        sid = lax.axis_index("s")
        wid = sid * NC + cid
        base = wid * ew

        def do_group(b, ch):
            hi = [pltpu.async_copy(src_hbm.at[pl.ds(b + j * ch, ch)],
                                   si_v.at[j].at[pl.ds(0, ch)], semi) for j in range(K)]
            hi += [pltpu.async_copy(dst_hbm.at[pl.ds(b + j * ch, ch)],
                                    di_v.at[j].at[pl.ds(0, ch)], semi) for j in range(K)]
            for hh in hi:
                hh.wait()
            hg = [pltpu.async_copy(h_hbm.at[si_v.at[j].at[pl.ds(0, ch)]],
                                   srow_v.at[pl.ds(j * ch, ch)], semg) for j in range(K)]
            hg += [pltpu.async_copy(h_hbm.at[di_v.at[j].at[pl.ds(0, ch)]],
                                    drow_v.at[pl.ds(j * ch, ch)], semg) for j in range(K)]
            for hh in hg:
                hh.wait()
            hw = [pltpu.async_copy(srow_v.at[pl.ds(j * ch, ch)],
                                   hs_hbm.at[pl.ds(b + j * ch, ch)], semw) for j in range(K)]
            hw += [pltpu.async_copy(drow_v.at[pl.ds(j * ch, ch)],
                                    hd_hbm.at[pl.ds(b + j * ch, ch)], semw) for j in range(K)]
            for hh in hw:
                hh.wait()

        @pl.loop(0, full_groups, step=G)
        def _(off):
            do_group(base + off, CH)

        if tail:
            do_group(base + full_groups, tail // K)

    return _sc_gather


@functools.cache
def _make_sc_degree():
    mesh = plsc.VectorSubcoreMesh(core_axis_name="c", subcore_axis_name="s")

    @functools.partial(
        pl.kernel,
        mesh=mesh,
        out_type=jax.ShapeDtypeStruct((NC, N, D), jnp.float32),
        scratch_types=[
            pltpu.VMEM((K, CH), jnp.int32),
            pltpu.VMEM((CH, D), jnp.float32),
            pltpu.VMEM_SHARED((N, D), jnp.float32),
            pltpu.SemaphoreType.DMA,
            pltpu.SemaphoreType.DMA,
        ],
    )
    def _sc_degree(dst_hbm, z_d_hbm, ones_hbm, deg_hbm,
                   di_v, ones_v, deg_sh, semi, semw):
        cid = lax.axis_index("c")
        sid = lax.axis_index("s")
        wid = sid * NC + cid
        base = wid * EW

        r0 = sid * ROWS_A
        pltpu.sync_copy(z_d_hbm.at[pl.ds(r0, ROWS_A)], deg_sh.at[pl.ds(r0, ROWS_A)])

        @pl.when(sid == 0)
        def _():
            t0 = NS * ROWS_A
            pltpu.sync_copy(z_d_hbm.at[pl.ds(t0, TAIL_ROWS)], deg_sh.at[pl.ds(t0, TAIL_ROWS)])

        pltpu.sync_copy(ones_hbm, ones_v)
        plsc.subcore_barrier()

        @pl.loop(0, EW, step=G)
        def _(off):
            b = base + off
            hi = [pltpu.async_copy(dst_hbm.at[pl.ds(b + j * CH, CH)], di_v.at[j], semi)
                  for j in range(K)]
            for hh in hi:
                hh.wait()
            hw = [pltpu.async_copy(ones_v, deg_sh.at[di_v.at[j]], semw, add=True)
                  for j in range(K)]
            for hh in hw:
                hh.wait()

        plsc.subcore_barrier()

        pltpu.sync_copy(deg_sh.at[pl.ds(r0, ROWS_A)], deg_hbm.at[cid].at[pl.ds(r0, ROWS_A)])

        @pl.when(sid == 0)
        def _():
            t0 = NS * ROWS_A
            pltpu.sync_copy(deg_sh.at[pl.ds(t0, TAIL_ROWS)], deg_hbm.at[cid].at[pl.ds(t0, TAIL_ROWS)])

    return _sc_degree


def _mlp_body(hs_ref, hd_ref, e_ref, ws_ref, wd_ref, we_ref, bc_ref, w2_ref, b2_ref, out_ref):
    u = (
        jnp.dot(hs_ref[...], ws_ref[...], preferred_element_type=jnp.float32)
        + jnp.dot(hd_ref[...], wd_ref[...], preferred_element_type=jnp.float32)
        + jnp.dot(e_ref[...], we_ref[...], preferred_element_type=jnp.float32)
        + bc_ref[...]
    )
    pre = u[:, :D]
    gate_in = u[:, D:]
    hidden = pre * jax.nn.sigmoid(pre)
    msg = jnp.dot(hidden, w2_ref[...], preferred_element_type=jnp.float32) + b2_ref[...]
    out_ref[...] = msg * jax.nn.sigmoid(gate_in)


def _edge_mlp(hs, hd, e, ws, wd, we, bc, w2, b2, block=4000):
    grid = (hs.shape[0] // block,)
    full = lambda shape: pl.BlockSpec(shape, lambda i: (0, 0))
    return pl.pallas_call(
        _mlp_body,
        grid=grid,
        in_specs=[
            pl.BlockSpec((block, D), lambda i: (i, 0)),
            pl.BlockSpec((block, D), lambda i: (i, 0)),
            pl.BlockSpec((block, ED), lambda i: (i, 0)),
            full((D, 2 * D)),
            full((D, 2 * D)),
            full((ED, 2 * D)),
            full((1, 2 * D)),
            full((D, D)),
            full((1, D)),
        ],
        out_specs=pl.BlockSpec((block, D), lambda i: (i, 0)),
        out_shape=jax.ShapeDtypeStruct((hs.shape[0], D), jnp.float32),
    )(hs, hd, e, ws, wd, we, bc, w2, b2)


@functools.cache
def _make_sc_scatter(ne):
    mesh = plsc.VectorSubcoreMesh(core_axis_name="c", subcore_axis_name="s")
    ew = ne // NW

    @functools.partial(
        pl.kernel,
        mesh=mesh,
        out_type=jax.ShapeDtypeStruct((NC, N, D), jnp.float32),
        scratch_types=[
            pltpu.VMEM((K, CHB), jnp.int32),
            pltpu.VMEM((GB, D), jnp.float32),
            pltpu.VMEM_SHARED((N, D), jnp.float32),
            pltpu.SemaphoreType.DMA,
            pltpu.SemaphoreType.DMA,
        ],
    )
    def _sc_scatter(msg_hbm, dst_hbm, z_d_hbm, agg_hbm, di_v, msg_v, acc_sh, semi, sema):
        cid = lax.axis_index("c")
        sid = lax.axis_index("s")
        wid = sid * NC + cid
        base = wid * ew

        r0 = sid * ROWS_A
        pltpu.sync_copy(z_d_hbm.at[pl.ds(r0, ROWS_A)], acc_sh.at[pl.ds(r0, ROWS_A)])

        @pl.when(sid == 0)
        def _():
            t0 = NS * ROWS_A
            pltpu.sync_copy(z_d_hbm.at[pl.ds(t0, TAIL_ROWS)], acc_sh.at[pl.ds(t0, TAIL_ROWS)])

        plsc.subcore_barrier()

        def do_group(b, ch):
            hi = [pltpu.async_copy(dst_hbm.at[pl.ds(b + j * ch, ch)],
                                   di_v.at[j].at[pl.ds(0, ch)], semi) for j in range(K)]
            hi += [pltpu.async_copy(msg_hbm.at[pl.ds(b + j * ch, ch)],
                                    msg_v.at[pl.ds(j * ch, ch)], semi) for j in range(K)]
            for hh in hi:
                hh.wait()
            ha = [pltpu.async_copy(msg_v.at[pl.ds(j * ch, ch)],
                                   acc_sh.at[di_v.at[j].at[pl.ds(0, ch)]], sema, add=True)
                  for j in range(K)]
            for hh in ha:
                hh.wait()

        full_groups = (ew // GB) * GB
        tail = ew - full_groups

        @pl.loop(0, full_groups, step=GB)
        def _(off):
            do_group(base + off, CHB)

        if tail:
            do_group(base + full_groups, tail // K)

        plsc.subcore_barrier()

        pltpu.sync_copy(acc_sh.at[pl.ds(r0, ROWS_A)], agg_hbm.at[cid].at[pl.ds(r0, ROWS_A)])

        @pl.when(sid == 0)
        def _():
            t0 = NS * ROWS_A
            pltpu.sync_copy(acc_sh.at[pl.ds(t0, TAIL_ROWS)], agg_hbm.at[cid].at[pl.ds(t0, TAIL_ROWS)])

    return _sc_scatter


def _node_body(h_ref, agga_ref, aggb_ref, deg_ref, wa_ref, wb_ref, bu_ref, out_ref):
    agg = agga_ref[0] + agga_ref[1] + aggb_ref[0] + aggb_ref[1]
    deg = deg_ref[0][:, :1] + deg_ref[1][:, :1]
    aggn = agg / jnp.maximum(deg, 1.0)
    z = (
        jnp.dot(h_ref[...], wa_ref[...], preferred_element_type=jnp.float32)
        + jnp.dot(aggn, wb_ref[...], preferred_element_type=jnp.float32)
        + bu_ref[...]
    )
    out_ref[...] = z * jax.nn.sigmoid(z)


def _node_update(h, aggpa, aggpb, degp, wa, wb, bu, block=2000):
    grid = (N // block,)
    full = lambda shape: pl.BlockSpec(shape, lambda i: tuple(0 for _ in shape))
    return pl.pallas_call(
        _node_body,
        grid=grid,
        in_specs=[
            pl.BlockSpec((block, D), lambda i: (i, 0)),
            pl.BlockSpec((NC, block, D), lambda i: (0, i, 0)),
            pl.BlockSpec((NC, block, D), lambda i: (0, i, 0)),
            pl.BlockSpec((NC, block, D), lambda i: (0, i, 0)),
            full((D, D)),
            full((D, D)),
            full((1, D)),
        ],
        out_specs=pl.BlockSpec((block, D), lambda i: (i, 0)),
        out_shape=jax.ShapeDtypeStruct((N, D), jnp.float32),
    )(h, aggpa, aggpb, degp, wa, wb, bu)


def kernel(h, edge_index, e, W1, b1, W2, b2, Wg, bg, Wu, bu):
    src = edge_index[0]
    dst = edge_index[1]

    ws = jnp.concatenate([W1[:D], Wg[:D]], axis=1)
    wd = jnp.concatenate([W1[D:2 * D], Wg[D:2 * D]], axis=1)
    we = jnp.concatenate([W1[2 * D:], Wg[2 * D:]], axis=1)
    bc = jnp.concatenate([b1, bg])[None, :]

    z_d = jnp.zeros((N, D), jnp.float32)
    ones = jnp.ones((CH, D), jnp.float32)

    E2 = E // 2
    src_a, src_b = src[:E2], src[E2:]
    dst_a, dst_b = dst[:E2], dst[E2:]
    e_a, e_b = e[:E2], e[E2:]

    gather = _make_sc_gather(E2)
    scatter = _make_sc_scatter(E2)

    hs_a, hd_a = gather(h, src_a, dst_a)
    hs_b, hd_b = gather(h, src_b, dst_b)
    degp = _make_sc_degree()(dst, z_d, ones)

    msg_a = _edge_mlp(hs_a, hd_a, e_a, ws, wd, we, bc, W2, b2[None, :])
    msg_b = _edge_mlp(hs_b, hd_b, e_b, ws, wd, we, bc, W2, b2[None, :])

    aggpa = scatter(msg_a, dst_a, z_d)
    aggpb = scatter(msg_b, dst_b, z_d)

    return _node_update(h, aggpa, aggpb, degp, Wu[:D], Wu[D:], bu[None, :])
